# Initial kernel scaffold; baseline (speedup 1.0000x reference)
import jax
import jax.numpy as jnp
from jax.experimental import pallas as pl


def kernel(x, edge_index, W1, as1, ad1, b1, W2, as2, ad2, b2, W3, as3, ad3, b3):
    raise NotImplementedError("write your pallas kernel here")


# speedup vs baseline: 47.7164x; 47.7164x over previous
import functools

import jax
import jax.numpy as jnp
from jax import lax
from jax.experimental import pallas as pl
from jax.experimental.pallas import tpu as pltpu
from jax.experimental.pallas import tpu_sc as plsc

_N = 10000
_NP = 10240
_E = 320000
_NC = 2
_NS = 16
_EPT = _E // (_NC * _NS)
_B = 80
_NCHUNK = _EPT // _B
_BR = 1000


def _leaky(v):
    return jnp.maximum(v, 0.2 * v)


def _bcast16(vec, idx16):
    return lax.gather(
        vec, idx16.reshape(16, 1),
        lax.GatherDimensionNumbers(
            offset_dims=(), collapsed_slice_dims=(0,), start_index_map=(0,)),
        (1,), mode=lax.GatherScatterMode.PROMISE_IN_BOUNDS)


def _row_width(heads, ch):
    return ((heads * ch + 2 * heads + 15) // 16) * 16


def _head_slices(ch):
    out = []
    c = 0
    while c < ch:
        out.append((c, min(16, ch - c)))
        c += 16
    return out


def _make_sc_scatter(heads, ch):
    hb = heads * ch
    r = _row_width(heads, ch)
    nr = _NP // _NS
    slices = _head_slices(ch)
    mesh = plsc.VectorSubcoreMesh(core_axis_name="c", subcore_axis_name="s",
                                  num_cores=_NC, num_subcores=_NS)

    @functools.partial(
        pl.kernel,
        out_type=jax.ShapeDtypeStruct((_NC, _NP, r), jnp.float32),
        mesh=mesh,
        scratch_types=[
            pltpu.VMEM((_B,), jnp.int32),
            pltpu.VMEM((_B,), jnp.int32),
            pltpu.VMEM((_B, r), jnp.float32),
            pltpu.VMEM((_B, 16), jnp.float32),
            pltpu.VMEM_SHARED((_NP, r), jnp.float32),
            pltpu.SemaphoreType.DMA,
            pltpu.SemaphoreType.DMA,
        ],
        compiler_params=pltpu.CompilerParams(use_tc_tiling_on_sc=False,
                                             needs_layout_passes=False),
    )
    def sc_fn(src_hbm, dst_hbm, haug_hbm, adst_hbm, zero_hbm, out_hbm,
              srcv, dstv, rows, adst_buf, acc, sem, sem2):
        c = lax.axis_index("c")
        s = lax.axis_index("s")
        r0 = s * nr
        pltpu.sync_copy(zero_hbm.at[pl.ds(r0, nr)], acc.at[pl.ds(r0, nr)])
        plsc.subcore_barrier()

        ebase = (c * _NS + s) * _EPT
        lane = jnp.arange(16, dtype=jnp.int32)

        def chunk(i, carry):
            off = ebase + i * _B
            pltpu.sync_copy(src_hbm.at[pl.ds(off, _B)], srcv)
            pltpu.sync_copy(dst_hbm.at[pl.ds(off, _B)], dstv)
            g1 = pltpu.async_copy(haug_hbm.at[srcv], rows, sem)
            g2 = pltpu.async_copy(adst_hbm.at[dstv], adst_buf, sem2)
            g1.wait()
            g2.wait()

            def group(g, carry2):
                row_ids = g * 16 + lane
                ps = []
                for hd in range(heads):
                    acol = jnp.full((16,), hb + hd, dtype=jnp.int32)
                    asrc = plsc.load_gather(rows, [row_ids, acol])
                    adst = plsc.load_gather(
                        adst_buf, [row_ids, jnp.full((16,), hd, dtype=jnp.int32)])
                    p = jnp.exp(_leaky(asrc + adst))
                    plsc.store_scatter(rows, [row_ids, acol], p)
                    ps.append(p)
                for ei in range(16):
                    er = g * 16 + ei
                    eidx = jnp.full((16,), ei, dtype=jnp.int32)
                    for hd in range(heads):
                        pb = _bcast16(ps[hd], eidx)
                        for (cs, nv) in slices:
                            col = hd * ch + cs
                            m = pb if nv == 16 else jnp.where(lane < nv, pb, 1.0)
                            rows[er, pl.ds(col, 16)] = rows[er, pl.ds(col, 16)] * m
                return carry2

            lax.fori_loop(0, _B // 16, group, 0)
            pltpu.sync_copy(rows, acc.at[dstv], add=True)
            return carry

        lax.fori_loop(0, _NCHUNK, chunk, 0)
        plsc.subcore_barrier()
        pltpu.sync_copy(acc.at[pl.ds(r0, nr)], out_hbm.at[c, pl.ds(r0, nr)])

    return sc_fn


_sc_scatter_128 = _make_sc_scatter(4, 32)
_sc_scatter_40 = _make_sc_scatter(1, 40)


def _prep_w(W, a_src, a_dst, heads, ch):
    hb = heads * ch
    r = _row_width(heads, ch)
    psrc = jnp.zeros((hb, heads), jnp.float32).at[
        jnp.arange(hb), jnp.arange(hb) // ch].set(a_src.reshape(-1))
    pdst = jnp.zeros((hb, heads), jnp.float32).at[
        jnp.arange(hb), jnp.arange(hb) // ch].set(a_dst.reshape(-1))
    proj = jnp.concatenate(
        [jnp.eye(hb, dtype=jnp.float32), psrc, pdst,
         jnp.zeros((hb, r - hb - 2 * heads), jnp.float32)], axis=1)
    return W.astype(jnp.float32) @ proj


def _tc_first(x, wp, heads, ch):
    hb = heads * ch
    r = wp.shape[1]
    k = x.shape[1]

    def body(x_ref, w_ref, o_ref, a_ref):
        o = jnp.dot(x_ref[...], w_ref[...], preferred_element_type=jnp.float32)
        o_ref[...] = o
        a_ref[...] = jnp.concatenate(
            [o[:, hb + heads:hb + 2 * heads],
             jnp.zeros((o.shape[0], 16 - heads), jnp.float32)], axis=1)

    return pl.pallas_call(
        body,
        grid=(_N // _BR,),
        in_specs=[pl.BlockSpec((_BR, k), lambda i: (i, 0)),
                  pl.BlockSpec((k, r), lambda i: (0, 0))],
        out_specs=[pl.BlockSpec((_BR, r), lambda i: (i, 0)),
                   pl.BlockSpec((_BR, 16), lambda i: (i, 0))],
        out_shape=[jax.ShapeDtypeStruct((_N, r), jnp.float32),
                   jax.ShapeDtypeStruct((_N, 16), jnp.float32)],
    )(x, wp)


def _tc_mid(part, haug, smat, b2d, wp, heads, ch, heads_n, ch_n):
    hb = heads * ch
    r = part.shape[2]
    hb_n = heads_n * ch_n
    r_n = wp.shape[1]

    def body(p_ref, h_ref, s_ref, b_ref, w_ref, o_ref, a_ref):
        tot = p_ref[0] + p_ref[1]
        asrc = h_ref[:, hb:hb + heads]
        adst = h_ref[:, hb + heads:hb + 2 * heads]
        pself = jnp.exp(_leaky(asrc + adst))
        num = tot[:, :hb] + jnp.dot(
            pself, s_ref[...], preferred_element_type=jnp.float32) * h_ref[:, :hb]
        den = jnp.dot(tot[:, hb:hb + heads] + pself, s_ref[...],
                      preferred_element_type=jnp.float32) + 1e-16
        xn = num / den + b_ref[...]
        xn = jnp.where(xn > 0, xn, jnp.exp(jnp.minimum(xn, 0.0)) - 1.0)
        o = jnp.dot(xn, w_ref[...], preferred_element_type=jnp.float32)
        o_ref[...] = o
        a_ref[...] = jnp.concatenate(
            [o[:, hb_n + heads_n:hb_n + 2 * heads_n],
             jnp.zeros((o.shape[0], 16 - heads_n), jnp.float32)], axis=1)

    return pl.pallas_call(
        body,
        grid=(_N // _BR,),
        in_specs=[pl.BlockSpec((_NC, _BR, r), lambda i: (0, i, 0)),
                  pl.BlockSpec((_BR, r), lambda i: (i, 0)),
                  pl.BlockSpec((heads, hb), lambda i: (0, 0)),
                  pl.BlockSpec((1, hb), lambda i: (0, 0)),
                  pl.BlockSpec((hb, r_n), lambda i: (0, 0))],
        out_specs=[pl.BlockSpec((_BR, r_n), lambda i: (i, 0)),
                   pl.BlockSpec((_BR, 16), lambda i: (i, 0))],
        out_shape=[jax.ShapeDtypeStruct((_N, r_n), jnp.float32),
                   jax.ShapeDtypeStruct((_N, 16), jnp.float32)],
    )(part, haug, smat, b2d, wp)


def _tc_fin(part, haug, b2d, cls):
    r = part.shape[2]

    def body(p_ref, h_ref, b_ref, o_ref):
        tot = p_ref[0] + p_ref[1]
        asrc = h_ref[:, cls:cls + 1]
        adst = h_ref[:, cls + 1:cls + 2]
        pself = jnp.exp(_leaky(asrc + adst))
        num = tot[:, :cls] + pself * h_ref[:, :cls]
        den = tot[:, cls:cls + 1] + pself + 1e-16
        logits = num / den + b_ref[...]
        m = jnp.max(logits, axis=1, keepdims=True)
        lse = jnp.log(jnp.sum(jnp.exp(logits - m), axis=1, keepdims=True)) + m
        o_ref[...] = logits - lse

    return pl.pallas_call(
        body,
        grid=(_N // _BR,),
        in_specs=[pl.BlockSpec((_NC, _BR, r), lambda i: (0, i, 0)),
                  pl.BlockSpec((_BR, r), lambda i: (i, 0)),
                  pl.BlockSpec((1, cls), lambda i: (0, 0))],
        out_specs=pl.BlockSpec((_BR, cls), lambda i: (i, 0)),
        out_shape=jax.ShapeDtypeStruct((_N, cls), jnp.float32),
    )(part, haug, b2d)


def kernel(x, edge_index, W1, as1, ad1, b1, W2, as2, ad2, b2, W3, as3, ad3, b3):
    src = edge_index[0]
    dst = edge_index[1]
    w1p = _prep_w(W1, as1, ad1, 4, 32)
    w2p = _prep_w(W2, as2, ad2, 4, 32)
    w3p = _prep_w(W3, as3, ad3, 1, 40)
    s4 = jnp.kron(jnp.eye(4, dtype=jnp.float32), jnp.ones((1, 32), jnp.float32))
    z144 = jnp.zeros((_NP, _row_width(4, 32)), jnp.float32)
    z48 = jnp.zeros((_NP, _row_width(1, 40)), jnp.float32)

    haug1, adst1 = _tc_first(x, w1p, 4, 32)
    part1 = _sc_scatter_128(src, dst, haug1, adst1, z144)
    haug2, adst2 = _tc_mid(part1, haug1, s4, b1.reshape(1, -1), w2p, 4, 32, 4, 32)
    part2 = _sc_scatter_128(src, dst, haug2, adst2, z144)
    haug3, adst3 = _tc_mid(part2, haug2, s4, b2.reshape(1, -1), w3p, 4, 32, 1, 40)
    part3 = _sc_scatter_40(src, dst, haug3, adst3, z48)
    return _tc_fin(part3, haug3, b3.reshape(1, -1), 40)


# speedup vs baseline: 64.9285x; 1.3607x over previous
import functools

import jax
import jax.numpy as jnp
from jax import lax
from jax.experimental import pallas as pl
from jax.experimental.pallas import tpu as pltpu
from jax.experimental.pallas import tpu_sc as plsc

_N = 10000
_NP = 10240
_E = 320000
_NC = 2
_NS = 16
_EPT = _E // (_NC * _NS)
_B = 80
_NCHUNK = _EPT // _B
_BR = 1000


def _leaky(v):
    return jnp.maximum(v, 0.2 * v)


def _bcast16(vec, idx16):
    return lax.gather(
        vec, idx16.reshape(16, 1),
        lax.GatherDimensionNumbers(
            offset_dims=(), collapsed_slice_dims=(0,), start_index_map=(0,)),
        (1,), mode=lax.GatherScatterMode.PROMISE_IN_BOUNDS)


def _row_width(heads, ch):
    return ((heads * ch + 2 * heads + 15) // 16) * 16


def _head_slices(ch):
    out = []
    c = 0
    while c < ch:
        out.append((c, min(16, ch - c)))
        c += 16
    return out


def _make_sc_scatter(heads, ch):
    hb = heads * ch
    r = _row_width(heads, ch)
    nr = _NP // _NS
    slices = _head_slices(ch)
    mesh = plsc.VectorSubcoreMesh(core_axis_name="c", subcore_axis_name="s",
                                  num_cores=_NC, num_subcores=_NS)

    @functools.partial(
        pl.kernel,
        out_type=jax.ShapeDtypeStruct((_NC, _NP, r), jnp.float32),
        mesh=mesh,
        scratch_types=[
            pltpu.VMEM((_B,), jnp.int32),
            pltpu.VMEM((_B,), jnp.int32),
            pltpu.VMEM((_B, r), jnp.float32),
            pltpu.VMEM((_B, 16), jnp.float32),
            pltpu.VMEM((_B,), jnp.int32),
            pltpu.VMEM((_B,), jnp.int32),
            pltpu.VMEM((_B, r), jnp.float32),
            pltpu.VMEM((_B, 16), jnp.float32),
            pltpu.VMEM_SHARED((_NP, r), jnp.float32),
            pltpu.SemaphoreType.DMA,
            pltpu.SemaphoreType.DMA,
            pltpu.SemaphoreType.DMA,
            pltpu.SemaphoreType.DMA,
        ],
        compiler_params=pltpu.CompilerParams(use_tc_tiling_on_sc=False,
                                             needs_layout_passes=False),
    )
    def sc_fn(src_hbm, dst_hbm, haug_hbm, adst_hbm, zero_hbm, out_hbm,
              srcv0, dstv0, rows0, adst0, srcv1, dstv1, rows1, adst1,
              acc, semh0, sema0, semh1, sema1):
        c = lax.axis_index("c")
        s = lax.axis_index("s")
        srcvs, dstvs = [srcv0, srcv1], [dstv0, dstv1]
        rowss, adsts = [rows0, rows1], [adst0, adst1]
        semhs, semas = [semh0, semh1], [sema0, sema1]
        r0 = s * nr
        pltpu.sync_copy(zero_hbm.at[pl.ds(r0, nr)], acc.at[pl.ds(r0, nr)])
        plsc.subcore_barrier()

        ebase = (c * _NS + s) * _EPT
        lane = jnp.arange(16, dtype=jnp.int32)

        def issue(i, k):
            off = ebase + i * _B
            pltpu.sync_copy(src_hbm.at[pl.ds(off, _B)], srcvs[k])
            pltpu.sync_copy(dst_hbm.at[pl.ds(off, _B)], dstvs[k])
            pltpu.async_copy(haug_hbm.at[srcvs[k]], rowss[k], semhs[k])
            pltpu.async_copy(adst_hbm.at[dstvs[k]], adsts[k], semas[k])

        def wait_gather(k):
            pltpu.make_async_copy(haug_hbm.at[srcvs[k]], rowss[k], semhs[k]).wait()
            pltpu.make_async_copy(adst_hbm.at[dstvs[k]], adsts[k], semas[k]).wait()

        def process(k):
            rows, adst_buf, dstv = rowss[k], adsts[k], dstvs[k]

            def group(g, carry2):
                row_ids = g * 16 + lane
                ps = []
                for hd in range(heads):
                    acol = jnp.full((16,), hb + hd, dtype=jnp.int32)
                    asrc = plsc.load_gather(rows, [row_ids, acol])
                    adst = plsc.load_gather(
                        adst_buf, [row_ids, jnp.full((16,), hd, dtype=jnp.int32)])
                    p = jnp.exp(_leaky(asrc + adst))
                    plsc.store_scatter(rows, [row_ids, acol], p)
                    ps.append(p)
                for ei in range(16):
                    er = g * 16 + ei
                    eidx = jnp.full((16,), ei, dtype=jnp.int32)
                    for hd in range(heads):
                        pb = _bcast16(ps[hd], eidx)
                        for (cs, nv) in slices:
                            col = hd * ch + cs
                            m = pb if nv == 16 else jnp.where(lane < nv, pb, 1.0)
                            rows[er, pl.ds(col, 16)] = rows[er, pl.ds(col, 16)] * m
                return carry2

            lax.fori_loop(0, _B // 16, group, 0)
            pltpu.sync_copy(rows, acc.at[dstv], add=True)

        issue(0, 0)

        def pair(j, carry):
            issue(2 * j + 1, 1)
            wait_gather(0)
            process(0)
            issue(2 * j + 2, 0)
            wait_gather(1)
            process(1)
            return carry

        lax.fori_loop(0, (_NCHUNK - 1) // 2, pair, 0)
        wait_gather(0)
        process(0)
        plsc.subcore_barrier()
        pltpu.sync_copy(acc.at[pl.ds(r0, nr)], out_hbm.at[c, pl.ds(r0, nr)])

    return sc_fn


_sc_scatter_128 = _make_sc_scatter(4, 32)
_sc_scatter_40 = _make_sc_scatter(1, 40)


def _prep_w(W, a_src, a_dst, heads, ch):
    hb = heads * ch
    r = _row_width(heads, ch)
    psrc = jnp.zeros((hb, heads), jnp.float32).at[
        jnp.arange(hb), jnp.arange(hb) // ch].set(a_src.reshape(-1))
    pdst = jnp.zeros((hb, heads), jnp.float32).at[
        jnp.arange(hb), jnp.arange(hb) // ch].set(a_dst.reshape(-1))
    proj = jnp.concatenate(
        [jnp.eye(hb, dtype=jnp.float32), psrc, pdst,
         jnp.zeros((hb, r - hb - 2 * heads), jnp.float32)], axis=1)
    return W.astype(jnp.float32) @ proj


def _tc_first(x, wp, heads, ch):
    hb = heads * ch
    r = wp.shape[1]
    k = x.shape[1]

    def body(x_ref, w_ref, o_ref, a_ref):
        o = jnp.dot(x_ref[...], w_ref[...], preferred_element_type=jnp.float32)
        o_ref[...] = o
        a_ref[...] = jnp.concatenate(
            [o[:, hb + heads:hb + 2 * heads],
             jnp.zeros((o.shape[0], 16 - heads), jnp.float32)], axis=1)

    return pl.pallas_call(
        body,
        grid=(_N // _BR,),
        in_specs=[pl.BlockSpec((_BR, k), lambda i: (i, 0)),
                  pl.BlockSpec((k, r), lambda i: (0, 0))],
        out_specs=[pl.BlockSpec((_BR, r), lambda i: (i, 0)),
                   pl.BlockSpec((_BR, 16), lambda i: (i, 0))],
        out_shape=[jax.ShapeDtypeStruct((_N, r), jnp.float32),
                   jax.ShapeDtypeStruct((_N, 16), jnp.float32)],
    )(x, wp)


def _tc_mid(part, haug, smat, b2d, wp, heads, ch, heads_n, ch_n):
    hb = heads * ch
    r = part.shape[2]
    hb_n = heads_n * ch_n
    r_n = wp.shape[1]

    def body(p_ref, h_ref, s_ref, b_ref, w_ref, o_ref, a_ref):
        tot = p_ref[0] + p_ref[1]
        asrc = h_ref[:, hb:hb + heads]
        adst = h_ref[:, hb + heads:hb + 2 * heads]
        pself = jnp.exp(_leaky(asrc + adst))
        num = tot[:, :hb] + jnp.dot(
            pself, s_ref[...], preferred_element_type=jnp.float32) * h_ref[:, :hb]
        den = jnp.dot(tot[:, hb:hb + heads] + pself, s_ref[...],
                      preferred_element_type=jnp.float32) + 1e-16
        xn = num / den + b_ref[...]
        xn = jnp.where(xn > 0, xn, jnp.exp(jnp.minimum(xn, 0.0)) - 1.0)
        o = jnp.dot(xn, w_ref[...], preferred_element_type=jnp.float32)
        o_ref[...] = o
        a_ref[...] = jnp.concatenate(
            [o[:, hb_n + heads_n:hb_n + 2 * heads_n],
             jnp.zeros((o.shape[0], 16 - heads_n), jnp.float32)], axis=1)

    return pl.pallas_call(
        body,
        grid=(_N // _BR,),
        in_specs=[pl.BlockSpec((_NC, _BR, r), lambda i: (0, i, 0)),
                  pl.BlockSpec((_BR, r), lambda i: (i, 0)),
                  pl.BlockSpec((heads, hb), lambda i: (0, 0)),
                  pl.BlockSpec((1, hb), lambda i: (0, 0)),
                  pl.BlockSpec((hb, r_n), lambda i: (0, 0))],
        out_specs=[pl.BlockSpec((_BR, r_n), lambda i: (i, 0)),
                   pl.BlockSpec((_BR, 16), lambda i: (i, 0))],
        out_shape=[jax.ShapeDtypeStruct((_N, r_n), jnp.float32),
                   jax.ShapeDtypeStruct((_N, 16), jnp.float32)],
    )(part, haug, smat, b2d, wp)


def _tc_fin(part, haug, b2d, cls):
    r = part.shape[2]

    def body(p_ref, h_ref, b_ref, o_ref):
        tot = p_ref[0] + p_ref[1]
        asrc = h_ref[:, cls:cls + 1]
        adst = h_ref[:, cls + 1:cls + 2]
        pself = jnp.exp(_leaky(asrc + adst))
        num = tot[:, :cls] + pself * h_ref[:, :cls]
        den = tot[:, cls:cls + 1] + pself + 1e-16
        logits = num / den + b_ref[...]
        m = jnp.max(logits, axis=1, keepdims=True)
        lse = jnp.log(jnp.sum(jnp.exp(logits - m), axis=1, keepdims=True)) + m
        o_ref[...] = logits - lse

    return pl.pallas_call(
        body,
        grid=(_N // _BR,),
        in_specs=[pl.BlockSpec((_NC, _BR, r), lambda i: (0, i, 0)),
                  pl.BlockSpec((_BR, r), lambda i: (i, 0)),
                  pl.BlockSpec((1, cls), lambda i: (0, 0))],
        out_specs=pl.BlockSpec((_BR, cls), lambda i: (i, 0)),
        out_shape=jax.ShapeDtypeStruct((_N, cls), jnp.float32),
    )(part, haug, b2d)


def kernel(x, edge_index, W1, as1, ad1, b1, W2, as2, ad2, b2, W3, as3, ad3, b3):
    src = edge_index[0]
    dst = edge_index[1]
    w1p = _prep_w(W1, as1, ad1, 4, 32)
    w2p = _prep_w(W2, as2, ad2, 4, 32)
    w3p = _prep_w(W3, as3, ad3, 1, 40)
    s4 = jnp.kron(jnp.eye(4, dtype=jnp.float32), jnp.ones((1, 32), jnp.float32))
    z144 = jnp.zeros((_NP, _row_width(4, 32)), jnp.float32)
    z48 = jnp.zeros((_NP, _row_width(1, 40)), jnp.float32)

    haug1, adst1 = _tc_first(x, w1p, 4, 32)
    part1 = _sc_scatter_128(src, dst, haug1, adst1, z144)
    haug2, adst2 = _tc_mid(part1, haug1, s4, b1.reshape(1, -1), w2p, 4, 32, 4, 32)
    part2 = _sc_scatter_128(src, dst, haug2, adst2, z144)
    haug3, adst3 = _tc_mid(part2, haug2, s4, b2.reshape(1, -1), w3p, 4, 32, 1, 40)
    part3 = _sc_scatter_40(src, dst, haug3, adst3, z48)
    return _tc_fin(part3, haug3, b3.reshape(1, -1), 40)
